# Initial kernel scaffold; baseline (speedup 1.0000x reference)
import jax
import jax.numpy as jnp
from jax.experimental import pallas as pl


def kernel(x, edge_index, batch, W_rel1, b_rel1, W_root1, W_rel2, b_rel2, W_root2):
    raise NotImplementedError("write your pallas kernel here")


# speedup vs baseline: 6.0952x; 6.0952x over previous
import functools

import jax
import jax.numpy as jnp
from jax import lax
from jax.experimental import pallas as pl
from jax.experimental.pallas import tpu as pltpu
from jax.experimental.pallas import tpu_sc as plsc

N = 10000
D = 128
G = 256
E = 320000

NC = 2
NS = 16
NW = NC * NS

C = 128
PER_TILE = ((E + NW * C - 1) // (NW * C)) * C
E_PAD = PER_TILE * NW
N_PAD = 10240
ZROWS = 128
ROWS_PER_TILE = N_PAD // NS

_PREC = jax.lax.Precision.HIGHEST


def _sc_segment_partials(feats, src_p, dst_p):
  mesh = plsc.VectorSubcoreMesh(core_axis_name="c", subcore_axis_name="s")

  @functools.partial(
      pl.kernel,
      out_type=jax.ShapeDtypeStruct((NC, N_PAD, D), jnp.float32),
      mesh=mesh,
      scratch_types=[
          pltpu.VMEM((C,), jnp.int32),
          pltpu.VMEM((C,), jnp.int32),
          pltpu.VMEM((C, D), jnp.float32),
          pltpu.VMEM((ZROWS, D), jnp.float32),
          pltpu.VMEM_SHARED((N_PAD, D), jnp.float32),
          pltpu.SemaphoreType.DMA,
      ],
  )
  def k(feats_hbm, src_hbm, dst_hbm, out_hbm, sidx, didx, rows, zbuf, accum, sem):
    cid = lax.axis_index("c")
    sid = lax.axis_index("s")
    wid = cid * NS + sid

    def zfill(i, carry):
      r = i // (D // 16)
      c0 = (i % (D // 16)) * 16
      zbuf[r, pl.ds(c0, 16)] = jnp.zeros((16,), jnp.float32)
      return carry

    lax.fori_loop(0, ZROWS * (D // 16), zfill, 0)

    def zcopy(j, carry):
      pltpu.sync_copy(zbuf, accum.at[pl.ds(sid * ROWS_PER_TILE + j * ZROWS, ZROWS)])
      return carry

    lax.fori_loop(0, ROWS_PER_TILE // ZROWS, zcopy, 0)
    plsc.subcore_barrier()

    base = wid * PER_TILE

    def chunk(i, carry):
      off = base + i * C
      pltpu.sync_copy(src_hbm.at[pl.ds(off, C)], sidx)
      pltpu.sync_copy(dst_hbm.at[pl.ds(off, C)], didx)
      pltpu.async_copy(feats_hbm.at[sidx], rows, sem).wait()
      pltpu.sync_copy(rows, accum.at[didx], add=True)
      return carry

    lax.fori_loop(0, PER_TILE // C, chunk, 0)
    plsc.subcore_barrier()

    pltpu.sync_copy(
        accum.at[pl.ds(sid * ROWS_PER_TILE, ROWS_PER_TILE)],
        out_hbm.at[cid, pl.ds(sid * ROWS_PER_TILE, ROWS_PER_TILE)],
    )

  return k(feats, src_p, dst_p)


def _tc_layer_body(p_ref, x_ref, wr_ref, b_ref, wt_ref, h_ref):
  agg = p_ref[0] + p_ref[1]
  h = lax.dot_general(agg, wr_ref[...], (((1,), (0,)), ((), ())),
                      precision=_PREC, preferred_element_type=jnp.float32)
  h = h + lax.dot_general(x_ref[...], wt_ref[...], (((1,), (0,)), ((), ())),
                          precision=_PREC, preferred_element_type=jnp.float32)
  h = h + b_ref[...]
  h_ref[...] = jnp.maximum(h, 0.0)


def _tc_layer(p, x, w_rel, b, w_root):
  R = 1000
  grid = N // R
  return pl.pallas_call(
      _tc_layer_body,
      grid=(grid,),
      in_specs=[
          pl.BlockSpec((NC, R, D), lambda i: (0, i, 0)),
          pl.BlockSpec((R, D), lambda i: (i, 0)),
          pl.BlockSpec((D, D), lambda i: (0, 0)),
          pl.BlockSpec((1, D), lambda i: (0, 0)),
          pl.BlockSpec((D, D), lambda i: (0, 0)),
      ],
      out_specs=pl.BlockSpec((R, D), lambda i: (i, 0)),
      out_shape=jax.ShapeDtypeStruct((N, D), jnp.float32),
  )(p, x, w_rel, b, w_root)


R2 = 400
GRID2 = N // R2


def _tc_layer_pool_body(q_ref, h_ref, wr_ref, b_ref, wt_ref, batch_ref,
                        out_ref, acc_ref, cnt_ref):
  step = pl.program_id(0)

  @pl.when(step == 0)
  def _():
    acc_ref[...] = jnp.zeros_like(acc_ref)
    cnt_ref[...] = jnp.zeros_like(cnt_ref)

  agg = q_ref[0] + q_ref[1]
  h2 = lax.dot_general(agg, wr_ref[...], (((1,), (0,)), ((), ())),
                       precision=_PREC, preferred_element_type=jnp.float32)
  h2 = h2 + lax.dot_general(h_ref[...], wt_ref[...], (((1,), (0,)), ((), ())),
                            precision=_PREC, preferred_element_type=jnp.float32)
  h2 = h2 + b_ref[...]
  h2 = jnp.maximum(h2, 0.0)

  bb = batch_ref[0, 0, :]
  iota_g = lax.broadcasted_iota(jnp.int32, (G, R2), 0)
  onehot = (iota_g == bb[None, :]).astype(jnp.float32)
  acc_ref[...] += lax.dot_general(onehot, h2, (((1,), (0,)), ((), ())),
                                  precision=_PREC,
                                  preferred_element_type=jnp.float32)
  cnt_ref[...] += jnp.broadcast_to(
      jnp.sum(onehot, axis=1, keepdims=True), (G, D))

  @pl.when(step == GRID2 - 1)
  def _():
    out_ref[...] = acc_ref[...] / jnp.maximum(cnt_ref[...], 1.0)


def _tc_layer_pool(q, h, w_rel, b, w_root, batch3d):
  return pl.pallas_call(
      _tc_layer_pool_body,
      grid=(GRID2,),
      in_specs=[
          pl.BlockSpec((NC, R2, D), lambda i: (0, i, 0)),
          pl.BlockSpec((R2, D), lambda i: (i, 0)),
          pl.BlockSpec((D, D), lambda i: (0, 0)),
          pl.BlockSpec((1, D), lambda i: (0, 0)),
          pl.BlockSpec((D, D), lambda i: (0, 0)),
          pl.BlockSpec((1, 1, R2), lambda i: (i, 0, 0)),
      ],
      out_specs=pl.BlockSpec((G, D), lambda i: (0, 0)),
      out_shape=jax.ShapeDtypeStruct((G, D), jnp.float32),
      scratch_shapes=[
          pltpu.VMEM((G, D), jnp.float32),
          pltpu.VMEM((G, D), jnp.float32),
      ],
  )(q, h, w_rel, b, w_root, batch3d)


def kernel(x, edge_index, batch, W_rel1, b_rel1, W_root1, W_rel2, b_rel2, W_root2):
  src = edge_index[0]
  dst = edge_index[1]
  npad = E_PAD - E
  pad_src = (jnp.arange(npad, dtype=jnp.int32) % N)
  pad_dst = N + (jnp.arange(npad, dtype=jnp.int32) % (N_PAD - N))
  src_p = jnp.concatenate([src, pad_src])
  dst_p = jnp.concatenate([dst, pad_dst])

  b1 = b_rel1.reshape(1, D)
  b2 = b_rel2.reshape(1, D)
  batch3d = batch.reshape(GRID2, 1, R2)

  p = _sc_segment_partials(x, src_p, dst_p)
  h = _tc_layer(p, x, W_rel1, b1, W_root1)
  q = _sc_segment_partials(h, src_p, dst_p)
  out = _tc_layer_pool(q, h, W_rel2, b2, W_root2, batch3d)
  return out


# speedup vs baseline: 11.2821x; 1.8510x over previous
import functools

import jax
import jax.numpy as jnp
from jax import lax
from jax.experimental import pallas as pl
from jax.experimental.pallas import tpu as pltpu
from jax.experimental.pallas import tpu_sc as plsc

N = 10000
D = 128
G = 256
E = 320000

NC = 2
NS = 16
NW = NC * NS

C = 128
NCHUNK = 80
HALF = NCHUNK // 2
PER_TILE = NCHUNK * C
E_PAD = PER_TILE * NW
N_PAD = 10240
ZROWS = 128
ROWS_PER_TILE = N_PAD // NS

_PREC = jax.lax.Precision.HIGHEST


def _sc_segment_partials(feats, edges_p):
  mesh = plsc.VectorSubcoreMesh(core_axis_name="c", subcore_axis_name="s")

  @functools.partial(
      pl.kernel,
      out_type=jax.ShapeDtypeStruct((NC, N_PAD, D), jnp.float32),
      mesh=mesh,
      scratch_types=[
          pltpu.VMEM((HALF, 2, C), jnp.int32),
          pltpu.VMEM((C, D), jnp.float32),
          pltpu.VMEM((C, D), jnp.float32),
          pltpu.VMEM_SHARED((N_PAD, D), jnp.float32),
          pltpu.SemaphoreType.DMA,
          pltpu.SemaphoreType.DMA,
      ],
  )
  def k(feats_hbm, edges_hbm, out_hbm, idx, rows0, rows1, accum, sem0, sem1):
    cid = lax.axis_index("c")
    sid = lax.axis_index("s")
    wid = cid * NS + sid

    def zfill(i, carry):
      r = i // (D // 16)
      c0 = (i % (D // 16)) * 16
      rows0[r, pl.ds(c0, 16)] = jnp.zeros((16,), jnp.float32)
      return carry

    lax.fori_loop(0, ZROWS * (D // 16), zfill, 0)

    def zcopy(j, carry):
      pltpu.sync_copy(rows0, accum.at[pl.ds(sid * ROWS_PER_TILE + j * ZROWS, ZROWS)])
      return carry

    lax.fori_loop(0, ROWS_PER_TILE // ZROWS, zcopy, 0)
    plsc.subcore_barrier()

    for h in range(NCHUNK // HALF):
      pltpu.sync_copy(edges_hbm.at[wid, pl.ds(h * HALF, HALF)], idx)
      pltpu.async_copy(feats_hbm.at[idx.at[0, 0]], rows0, sem0)
      pltpu.async_copy(feats_hbm.at[idx.at[1, 0]], rows1, sem1)

      def chunk(i, carry):
        i0 = 2 * i
        pltpu.make_async_copy(feats_hbm.at[idx.at[i0, 0]], rows0, sem0).wait()
        pltpu.sync_copy(rows0, accum.at[idx.at[i0, 1]], add=True)

        @pl.when(i0 + 2 < HALF)
        def _():
          pltpu.async_copy(feats_hbm.at[idx.at[i0 + 2, 0]], rows0, sem0)

        pltpu.make_async_copy(feats_hbm.at[idx.at[i0 + 1, 0]], rows1, sem1).wait()
        pltpu.sync_copy(rows1, accum.at[idx.at[i0 + 1, 1]], add=True)

        @pl.when(i0 + 3 < HALF)
        def _():
          pltpu.async_copy(feats_hbm.at[idx.at[i0 + 3, 0]], rows1, sem1)

        return carry

      lax.fori_loop(0, HALF // 2, chunk, 0)
    plsc.subcore_barrier()

    pltpu.sync_copy(
        accum.at[pl.ds(sid * ROWS_PER_TILE, ROWS_PER_TILE)],
        out_hbm.at[cid, pl.ds(sid * ROWS_PER_TILE, ROWS_PER_TILE)],
    )

  return k(feats, edges_p)


def _tc_layer_body(p_ref, x_ref, wr_ref, b_ref, wt_ref, h_ref):
  agg = p_ref[0] + p_ref[1]
  h = lax.dot_general(agg, wr_ref[...], (((1,), (0,)), ((), ())),
                      precision=_PREC, preferred_element_type=jnp.float32)
  h = h + lax.dot_general(x_ref[...], wt_ref[...], (((1,), (0,)), ((), ())),
                          precision=_PREC, preferred_element_type=jnp.float32)
  h = h + b_ref[...]
  h_ref[...] = jnp.maximum(h, 0.0)


def _tc_layer(p, x, w_rel, b, w_root):
  R = 1000
  grid = N // R
  return pl.pallas_call(
      _tc_layer_body,
      grid=(grid,),
      in_specs=[
          pl.BlockSpec((NC, R, D), lambda i: (0, i, 0)),
          pl.BlockSpec((R, D), lambda i: (i, 0)),
          pl.BlockSpec((D, D), lambda i: (0, 0)),
          pl.BlockSpec((1, D), lambda i: (0, 0)),
          pl.BlockSpec((D, D), lambda i: (0, 0)),
      ],
      out_specs=pl.BlockSpec((R, D), lambda i: (i, 0)),
      out_shape=jax.ShapeDtypeStruct((N, D), jnp.float32),
  )(p, x, w_rel, b, w_root)


R2 = 400
GRID2 = N // R2


def _tc_layer_pool_body(q_ref, h_ref, wr_ref, b_ref, wt_ref, batch_ref,
                        out_ref, acc_ref, cnt_ref):
  step = pl.program_id(0)

  @pl.when(step == 0)
  def _():
    acc_ref[...] = jnp.zeros_like(acc_ref)
    cnt_ref[...] = jnp.zeros_like(cnt_ref)

  agg = q_ref[0] + q_ref[1]
  h2 = lax.dot_general(agg, wr_ref[...], (((1,), (0,)), ((), ())),
                       precision=_PREC, preferred_element_type=jnp.float32)
  h2 = h2 + lax.dot_general(h_ref[...], wt_ref[...], (((1,), (0,)), ((), ())),
                            precision=_PREC, preferred_element_type=jnp.float32)
  h2 = h2 + b_ref[...]
  h2 = jnp.maximum(h2, 0.0)

  bb = batch_ref[0, 0, :]
  iota_g = lax.broadcasted_iota(jnp.int32, (G, R2), 0)
  onehot = (iota_g == bb[None, :]).astype(jnp.float32)
  acc_ref[...] += lax.dot_general(onehot, h2, (((1,), (0,)), ((), ())),
                                  precision=_PREC,
                                  preferred_element_type=jnp.float32)
  cnt_ref[...] += jnp.broadcast_to(
      jnp.sum(onehot, axis=1, keepdims=True), (G, D))

  @pl.when(step == GRID2 - 1)
  def _():
    out_ref[...] = acc_ref[...] / jnp.maximum(cnt_ref[...], 1.0)


def _tc_layer_pool(q, h, w_rel, b, w_root, batch3d):
  return pl.pallas_call(
      _tc_layer_pool_body,
      grid=(GRID2,),
      in_specs=[
          pl.BlockSpec((NC, R2, D), lambda i: (0, i, 0)),
          pl.BlockSpec((R2, D), lambda i: (i, 0)),
          pl.BlockSpec((D, D), lambda i: (0, 0)),
          pl.BlockSpec((1, D), lambda i: (0, 0)),
          pl.BlockSpec((D, D), lambda i: (0, 0)),
          pl.BlockSpec((1, 1, R2), lambda i: (i, 0, 0)),
      ],
      out_specs=pl.BlockSpec((G, D), lambda i: (0, 0)),
      out_shape=jax.ShapeDtypeStruct((G, D), jnp.float32),
      scratch_shapes=[
          pltpu.VMEM((G, D), jnp.float32),
          pltpu.VMEM((G, D), jnp.float32),
      ],
  )(q, h, w_rel, b, w_root, batch3d)


def kernel(x, edge_index, batch, W_rel1, b_rel1, W_root1, W_rel2, b_rel2, W_root2):
  src = edge_index[0]
  dst = edge_index[1]
  npad = E_PAD - E
  pad_src = (jnp.arange(npad, dtype=jnp.int32) % N)
  pad_dst = N + (jnp.arange(npad, dtype=jnp.int32) % (N_PAD - N))
  src_p = jnp.concatenate([src, pad_src]).reshape(NW, NCHUNK, 1, C)
  dst_p = jnp.concatenate([dst, pad_dst]).reshape(NW, NCHUNK, 1, C)
  edges_p = jnp.concatenate([src_p, dst_p], axis=2)

  b1 = b_rel1.reshape(1, D)
  b2 = b_rel2.reshape(1, D)
  batch3d = batch.reshape(GRID2, 1, R2)

  p = _sc_segment_partials(x, edges_p)
  h = _tc_layer(p, x, W_rel1, b1, W_root1)
  q = _sc_segment_partials(h, edges_p)
  out = _tc_layer_pool(q, h, W_rel2, b2, W_root2, batch3d)
  return out
